# baseline (speedup 1.0000x reference)
import jax
import jax.numpy as jnp
from jax import lax
from jax.experimental import pallas as pl
from jax.experimental.pallas import tpu as pltpu
from jax.experimental.pallas import tpu_sc as plsc

N = 10000
NP = 10240
E = 320000
EROWS = 2560
EP = EROWS * 128
B, S = 16, 2048
T = B * S
TROWS = T // 128
D = 128
NC, NS = 2, 16
NW = NC * NS
RPT = EROWS // NW
NSLICE = NP // NS

_mesh = lambda: plsc.VectorSubcoreMesh(
    core_axis_name="c", subcore_axis_name="s", num_cores=NC, num_subcores=NS)


def _deg_body(dst_hbm, w_hbm, zflat_hbm, deg_out, idxb, wb, sv, degsp):
    cid = lax.axis_index("c")
    sid = lax.axis_index("s")
    wid = cid * NS + sid
    pltpu.sync_copy(zflat_hbm, sv)
    pltpu.sync_copy(sv, degsp.at[pl.ds(sid * NSLICE, NSLICE)])
    plsc.subcore_barrier()

    def chunk(i, _):
        r0 = wid * RPT + i * 16
        pltpu.sync_copy(dst_hbm.at[pl.ds(r0, 16)], idxb)
        pltpu.sync_copy(w_hbm.at[pl.ds(r0, 16)], wb)
        for j in range(16):
            pltpu.sync_copy(wb.at[j], degsp.at[idxb.at[j]], add=True)
        return _

    lax.fori_loop(0, RPT // 16, chunk, None)
    plsc.subcore_barrier()
    pltpu.sync_copy(degsp.at[pl.ds(sid * NSLICE, NSLICE)], sv)
    pltpu.sync_copy(sv, deg_out.at[cid].at[pl.ds(sid * NSLICE, NSLICE)])


def _deg_call(dst2d, w2d, zflat):
    return pl.kernel(
        _deg_body,
        out_type=jax.ShapeDtypeStruct((NC, NP), jnp.float32),
        mesh=_mesh(),
        scratch_types=[
            pltpu.VMEM((16, 128), jnp.int32),
            pltpu.VMEM((16, 128), jnp.float32),
            pltpu.VMEM((NSLICE,), jnp.float32),
            pltpu.VMEM_SHARED((NP,), jnp.float32),
        ],
    )(dst2d, w2d, zflat)


GR = 80
CH = 4
EP2 = 327680
ER2 = EP2 // GR
GPT = ER2 // NW
ACCR = 10112
ASL = ACCR // NS


def _msg_body(src_hbm, dst_hbm, w_hbm, g_hbm, zrow_hbm, acc_out,
              srcb, dstb, wb, rbf, sbuf, gsem, ssem, accsp):
    cid = lax.axis_index("c")
    sid = lax.axis_index("s")
    wid = cid * NS + sid
    pltpu.sync_copy(zrow_hbm, accsp.at[pl.ds(sid * ASL, ASL)])
    plsc.subcore_barrier()

    def chunk(i, _):
        r0 = wid * GPT + i * CH
        pltpu.sync_copy(src_hbm.at[pl.ds(r0, CH)], srcb)
        pltpu.sync_copy(dst_hbm.at[pl.ds(r0, CH)], dstb)
        pltpu.sync_copy(w_hbm.at[pl.ds(r0, CH)], wb)
        gd = [None] * CH
        sd = [None] * CH
        gd[0] = pltpu.async_copy(g_hbm.at[srcb.at[0]], rbf.at[0], gsem[0])
        for j in range(CH):
            b = j % 2
            if j + 1 < CH:
                gd[j + 1] = pltpu.async_copy(
                    g_hbm.at[srcb.at[j + 1]], rbf.at[(j + 1) % 2],
                    gsem[(j + 1) % 2])
            gd[j].wait()
            if j >= 2:
                sd[j - 2].wait()

            def scale_rows(gidx, _c):
                wv = wb[j, pl.ds(gidx * 16, 16)]
                for k in range(16):
                    bw = lax.broadcast(wv[k], (16,))
                    r = gidx * 16 + k
                    for q in range(8):
                        sl = pl.ds(q * 16, 16)
                        sbuf[b, r, sl] = rbf[b, r, sl] * bw
                return _c

            lax.fori_loop(0, GR // 16, scale_rows, None)
            sd[j] = pltpu.async_copy(sbuf.at[b], accsp.at[dstb.at[j]],
                                     ssem[b], add=True)
        sd[CH - 2].wait()
        sd[CH - 1].wait()
        return _

    lax.fori_loop(0, GPT // CH, chunk, None)
    plsc.subcore_barrier()
    pltpu.sync_copy(accsp.at[pl.ds(sid * ASL, ASL)],
                    acc_out.at[cid].at[pl.ds(sid * ASL, ASL)])


def _msg_call(src96, dst96, w96, g, zrow):
    return pl.kernel(
        _msg_body,
        out_type=jax.ShapeDtypeStruct((NC, NP, D), jnp.float32),
        mesh=_mesh(),
        scratch_types=[
            pltpu.VMEM((CH, GR), jnp.int32),
            pltpu.VMEM((CH, GR), jnp.int32),
            pltpu.VMEM((CH, GR), jnp.float32),
            pltpu.VMEM((2, GR, D), jnp.float32),
            pltpu.VMEM((2, GR, D), jnp.float32),
            [pltpu.SemaphoreType.DMA] * 2,
            [pltpu.SemaphoreType.DMA] * 2,
            pltpu.VMEM_SHARED((ACCR, D), jnp.float32),
        ],
    )(src96, dst96, w96, g, zrow)


def _traj_body(emb_hbm, idx_hbm, out_hbm, idxb, rows, gsem, wsem):
    cid = lax.axis_index("c")
    sid = lax.axis_index("s")
    wid = cid * NS + sid
    rpt = TROWS // NW
    pltpu.sync_copy(idx_hbm.at[pl.ds(wid * rpt, rpt)], idxb)
    nb = 4
    gd = [None] * rpt
    wd = [None] * rpt
    for j in range(nb):
        gd[j] = pltpu.async_copy(emb_hbm.at[idxb.at[j]], rows.at[j], gsem[j])
    for j in range(rpt):
        b = j % nb
        gd[j].wait()
        if nb <= j + 1 < rpt:
            wd[j + 1 - nb].wait()
            gd[j + 1] = pltpu.async_copy(
                emb_hbm.at[idxb.at[j + 1]], rows.at[(j + 1) % nb],
                gsem[(j + 1) % nb])
        wd[j] = pltpu.async_copy(
            rows.at[b], out_hbm.at[pl.ds((wid * rpt + j) * 128, 128)], wsem[b])
    for j in range(rpt - nb, rpt):
        wd[j].wait()


def _traj_call(emb, idx2d):
    return pl.kernel(
        _traj_body,
        out_type=jax.ShapeDtypeStruct((T, D), jnp.float32),
        mesh=_mesh(),
        scratch_types=[
            pltpu.VMEM((TROWS // NW, 128), jnp.int32),
            pltpu.VMEM((4, 128, D), jnp.float32),
            [pltpu.SemaphoreType.DMA] * 4,
            [pltpu.SemaphoreType.DMA] * 4,
        ],
    )(emb, idx2d)


def _mmg_body(x_ref, w_ref, deg_ref, h_ref, dis_ref, g_ref):
    h = jnp.dot(x_ref[...], w_ref[...], preferred_element_type=jnp.float32)
    deg = 1.0 + deg_ref[0] + deg_ref[1]
    dis = lax.rsqrt(deg)
    h_ref[...] = h
    dis_ref[...] = dis
    g_ref[...] = h * dis


def _mmg_call(x_p, W, degp):
    blk = NP // 8
    return pl.pallas_call(
        _mmg_body,
        grid=(8,),
        in_specs=[
            pl.BlockSpec((blk, D), lambda i: (i, 0)),
            pl.BlockSpec((D, D), lambda i: (0, 0)),
            pl.BlockSpec((NC, blk, 1), lambda i: (0, i, 0)),
        ],
        out_specs=[
            pl.BlockSpec((blk, D), lambda i: (i, 0)),
            pl.BlockSpec((blk, 1), lambda i: (i, 0)),
            pl.BlockSpec((blk, D), lambda i: (i, 0)),
        ],
        out_shape=[
            jax.ShapeDtypeStruct((NP, D), jnp.float32),
            jax.ShapeDtypeStruct((NP, 1), jnp.float32),
            jax.ShapeDtypeStruct((NP, D), jnp.float32),
        ],
    )(x_p, W, degp)


def _emb_body(dis_ref, h_ref, acc_ref, b_ref, o_ref):
    i = pl.program_id(0)
    blk = NP // 8
    dis = dis_ref[...]
    s = acc_ref[0] + acc_ref[1]
    v = dis * s + (dis * dis) * h_ref[...] + b_ref[...]
    v = jnp.maximum(v, 0.0)
    row = i * blk + lax.broadcasted_iota(jnp.int32, (blk, D), 0)
    o_ref[...] = jnp.where(row < N, v, 0.0)


def _emb_call(dis, h_p, acc, b2):
    blk = NP // 8
    return pl.pallas_call(
        _emb_body,
        grid=(8,),
        in_specs=[
            pl.BlockSpec((blk, 1), lambda i: (i, 0)),
            pl.BlockSpec((blk, D), lambda i: (i, 0)),
            pl.BlockSpec((NC, blk, D), lambda i: (0, i, 0)),
            pl.BlockSpec((1, D), lambda i: (0, 0)),
        ],
        out_specs=pl.BlockSpec((blk, D), lambda i: (i, 0)),
        out_shape=jax.ShapeDtypeStruct((NP, D), jnp.float32),
    )(dis, h_p, acc, b2)


def kernel(x, edge_index, edge_attr, traj_seqs, W, b):
    src = edge_index[0].astype(jnp.int32)
    dst = edge_index[1].astype(jnp.int32)
    w = edge_attr.astype(jnp.float32)
    pad = EP - E
    spread = (jnp.arange(pad, dtype=jnp.int32) % (NP - N)) + N
    dst2d = jnp.concatenate([dst, spread]).reshape(EROWS, 128)
    w2d = jnp.pad(w, (0, pad)).reshape(EROWS, 128)
    pad2 = EP2 - E
    spread2 = jnp.arange(pad2, dtype=jnp.int32) % N
    src96 = jnp.concatenate([src, spread2]).reshape(ER2, GR)
    dst96 = jnp.concatenate([dst, spread2]).reshape(ER2, GR)
    w96 = jnp.pad(w, (0, pad2)).reshape(ER2, GR)

    x_p = jnp.pad(x, ((0, NP - N), (0, 0)))
    zflat = jnp.zeros((NSLICE,), jnp.float32)
    zrow = jnp.zeros((ASL, D), jnp.float32)
    b2 = b.reshape(1, D).astype(jnp.float32)

    traj = traj_seqs.astype(jnp.int32)
    is_pad = (traj < 0).astype(jnp.int32)
    mask = jnp.cumsum(is_pad, axis=1) == 0
    zrows = (jnp.arange(T, dtype=jnp.int32) % (NP - N)).reshape(B, S) + N
    idxsel = jnp.where(mask, jnp.clip(traj, 0, N - 1), zrows)
    idx2d = idxsel.reshape(TROWS, 128)

    degp = _deg_call(dst2d, w2d, zflat).reshape(NC, NP, 1)
    h_p, dis, g = _mmg_call(x_p, W, degp)
    acc = _msg_call(src96, dst96, w96, g, zrow)
    emb = _emb_call(dis, h_p, acc, b2)
    out = _traj_call(emb, idx2d)

    return out.reshape(B, S, D), mask


# speedup vs baseline: 1.3102x; 1.3102x over previous
import jax
import jax.numpy as jnp
from jax import lax
from jax.experimental import pallas as pl
from jax.experimental.pallas import tpu as pltpu
from jax.experimental.pallas import tpu_sc as plsc

N = 10000
NP = 10240
E = 320000
EROWS = 2560
EP = EROWS * 128
B, S = 16, 2048
T = B * S
TROWS = T // 128
D = 128
NC, NS = 2, 16
NW = NC * NS
RPT = EROWS // NW
NSLICE = NP // NS

_mesh = lambda: plsc.VectorSubcoreMesh(
    core_axis_name="c", subcore_axis_name="s", num_cores=NC, num_subcores=NS)


def _deg_body(dst_hbm, w_hbm, zflat_hbm, deg_out, idxb, wb, sv, degsp):
    cid = lax.axis_index("c")
    sid = lax.axis_index("s")
    wid = cid * NS + sid
    pltpu.sync_copy(zflat_hbm, sv)
    pltpu.sync_copy(sv, degsp.at[pl.ds(sid * NSLICE, NSLICE)])
    plsc.subcore_barrier()

    def chunk(i, _):
        r0 = wid * RPT + i * 16
        pltpu.sync_copy(dst_hbm.at[pl.ds(r0, 16)], idxb)
        pltpu.sync_copy(w_hbm.at[pl.ds(r0, 16)], wb)
        for j in range(16):
            pltpu.sync_copy(wb.at[j], degsp.at[idxb.at[j]], add=True)
        return _

    lax.fori_loop(0, RPT // 16, chunk, None)
    plsc.subcore_barrier()
    pltpu.sync_copy(degsp.at[pl.ds(sid * NSLICE, NSLICE)], sv)
    pltpu.sync_copy(sv, deg_out.at[cid].at[pl.ds(sid * NSLICE, NSLICE)])


def _deg_call(dst2d, w2d, zflat):
    return pl.kernel(
        _deg_body,
        out_type=jax.ShapeDtypeStruct((NC, NP), jnp.float32),
        mesh=_mesh(),
        scratch_types=[
            pltpu.VMEM((16, 128), jnp.int32),
            pltpu.VMEM((16, 128), jnp.float32),
            pltpu.VMEM((NSLICE,), jnp.float32),
            pltpu.VMEM_SHARED((NP,), jnp.float32),
        ],
    )(dst2d, w2d, zflat)


NB = 2
GR = 128
GPT = EROWS // NW


def _msg_body(src_hbm, dst_hbm, w_hbm, g_hbm, zrow_hbm, acc_out,
              srcb, dstb, wb, rows, gsem, ssem, accsp):
    cid = lax.axis_index("c")
    sid = lax.axis_index("s")
    wid = cid * NS + sid
    pltpu.sync_copy(zrow_hbm, accsp.at[pl.ds(sid * NSLICE, NSLICE)])
    plsc.subcore_barrier()

    def chunk(i, _):
        r0 = wid * GPT + i * 8
        pltpu.sync_copy(src_hbm.at[pl.ds(r0, 8)], srcb)
        pltpu.sync_copy(dst_hbm.at[pl.ds(r0, 8)], dstb)
        pltpu.sync_copy(w_hbm.at[pl.ds(r0, 8)], wb)
        gd = [None] * 8
        sd = [None] * 8
        for j in range(NB):
            gd[j] = pltpu.async_copy(g_hbm.at[srcb.at[j]], rows.at[j], gsem[j])
        for j in range(8):
            b = j % NB
            gd[j].wait()
            if NB <= j + 1 < 8:
                sd[j + 1 - NB].wait()
                gd[j + 1] = pltpu.async_copy(
                    g_hbm.at[srcb.at[j + 1]], rows.at[(j + 1) % NB],
                    gsem[(j + 1) % NB])

            def scale_rows(gidx, _c):
                wv = wb[j, pl.ds(gidx * 16, 16)]
                for k in range(16):
                    bw = lax.broadcast(wv[k], (16,))
                    r = gidx * 16 + k
                    for q in range(8):
                        sl = pl.ds(q * 16, 16)
                        rows[b, r, sl] = rows[b, r, sl] * bw
                return _c

            lax.fori_loop(0, GR // 16, scale_rows, None)
            sd[j] = pltpu.async_copy(rows.at[b], accsp.at[dstb.at[j]],
                                     ssem[b], add=True)
        for j in range(8 - NB, 8):
            sd[j].wait()
        return _

    lax.fori_loop(0, GPT // 8, chunk, None)
    plsc.subcore_barrier()
    pltpu.sync_copy(accsp.at[pl.ds(sid * NSLICE, NSLICE)],
                    acc_out.at[cid].at[pl.ds(sid * NSLICE, NSLICE)])


def _msg_call(src2d, dst2d2, w2d2, g, zrow):
    return pl.kernel(
        _msg_body,
        out_type=jax.ShapeDtypeStruct((NC, NP, D), jnp.float32),
        mesh=_mesh(),
        scratch_types=[
            pltpu.VMEM((8, GR), jnp.int32),
            pltpu.VMEM((8, GR), jnp.int32),
            pltpu.VMEM((8, GR), jnp.float32),
            pltpu.VMEM((NB, GR, D), jnp.float32),
            [pltpu.SemaphoreType.DMA] * NB,
            [pltpu.SemaphoreType.DMA] * NB,
            pltpu.VMEM_SHARED((NP, D), jnp.float32),
        ],
    )(src2d, dst2d2, w2d2, g, zrow)


def _traj_body(emb_hbm, idx_hbm, out_hbm, idxb, rows, gsem, wsem):
    cid = lax.axis_index("c")
    sid = lax.axis_index("s")
    wid = cid * NS + sid
    rpt = TROWS // NW
    pltpu.sync_copy(idx_hbm.at[pl.ds(wid * rpt, rpt)], idxb)
    nb = 4
    gd = [None] * rpt
    wd = [None] * rpt
    for j in range(nb):
        gd[j] = pltpu.async_copy(emb_hbm.at[idxb.at[j]], rows.at[j], gsem[j])
    for j in range(rpt):
        b = j % nb
        gd[j].wait()
        if nb <= j + 1 < rpt:
            wd[j + 1 - nb].wait()
            gd[j + 1] = pltpu.async_copy(
                emb_hbm.at[idxb.at[j + 1]], rows.at[(j + 1) % nb],
                gsem[(j + 1) % nb])
        wd[j] = pltpu.async_copy(
            rows.at[b], out_hbm.at[pl.ds((wid * rpt + j) * 128, 128)], wsem[b])
    for j in range(rpt - nb, rpt):
        wd[j].wait()


def _traj_call(emb, idx2d):
    return pl.kernel(
        _traj_body,
        out_type=jax.ShapeDtypeStruct((T, D), jnp.float32),
        mesh=_mesh(),
        scratch_types=[
            pltpu.VMEM((TROWS // NW, 128), jnp.int32),
            pltpu.VMEM((4, 128, D), jnp.float32),
            [pltpu.SemaphoreType.DMA] * 4,
            [pltpu.SemaphoreType.DMA] * 4,
        ],
    )(emb, idx2d)


def _mmg_body(x_ref, w_ref, deg_ref, h_ref, dis_ref, g_ref):
    h = jnp.dot(x_ref[...], w_ref[...], preferred_element_type=jnp.float32)
    deg = 1.0 + deg_ref[0] + deg_ref[1]
    dis = lax.rsqrt(deg)
    h_ref[...] = h
    dis_ref[...] = dis
    g_ref[...] = h * dis


def _mmg_call(x_p, W, degp):
    blk = NP // 8
    return pl.pallas_call(
        _mmg_body,
        grid=(8,),
        in_specs=[
            pl.BlockSpec((blk, D), lambda i: (i, 0)),
            pl.BlockSpec((D, D), lambda i: (0, 0)),
            pl.BlockSpec((NC, blk, 1), lambda i: (0, i, 0)),
        ],
        out_specs=[
            pl.BlockSpec((blk, D), lambda i: (i, 0)),
            pl.BlockSpec((blk, 1), lambda i: (i, 0)),
            pl.BlockSpec((blk, D), lambda i: (i, 0)),
        ],
        out_shape=[
            jax.ShapeDtypeStruct((NP, D), jnp.float32),
            jax.ShapeDtypeStruct((NP, 1), jnp.float32),
            jax.ShapeDtypeStruct((NP, D), jnp.float32),
        ],
    )(x_p, W, degp)


def _emb_body(dis_ref, h_ref, acc_ref, b_ref, o_ref):
    i = pl.program_id(0)
    blk = NP // 8
    dis = dis_ref[...]
    s = acc_ref[0] + acc_ref[1]
    v = dis * s + (dis * dis) * h_ref[...] + b_ref[...]
    v = jnp.maximum(v, 0.0)
    row = i * blk + lax.broadcasted_iota(jnp.int32, (blk, D), 0)
    o_ref[...] = jnp.where(row < N, v, 0.0)


def _emb_call(dis, h_p, acc, b2):
    blk = NP // 8
    return pl.pallas_call(
        _emb_body,
        grid=(8,),
        in_specs=[
            pl.BlockSpec((blk, 1), lambda i: (i, 0)),
            pl.BlockSpec((blk, D), lambda i: (i, 0)),
            pl.BlockSpec((NC, blk, D), lambda i: (0, i, 0)),
            pl.BlockSpec((1, D), lambda i: (0, 0)),
        ],
        out_specs=pl.BlockSpec((blk, D), lambda i: (i, 0)),
        out_shape=jax.ShapeDtypeStruct((NP, D), jnp.float32),
    )(dis, h_p, acc, b2)


def kernel(x, edge_index, edge_attr, traj_seqs, W, b):
    src = edge_index[0].astype(jnp.int32)
    dst = edge_index[1].astype(jnp.int32)
    w = edge_attr.astype(jnp.float32)
    pad = EP - E
    spread = (jnp.arange(pad, dtype=jnp.int32) % (NP - N)) + N
    src2d = jnp.concatenate([src, spread]).reshape(EROWS, 128)
    dst2d = jnp.concatenate([dst, spread]).reshape(EROWS, 128)
    w2d = jnp.pad(w, (0, pad)).reshape(EROWS, 128)

    x_p = jnp.pad(x, ((0, NP - N), (0, 0)))
    zflat = jnp.zeros((NSLICE,), jnp.float32)
    zrow = jnp.zeros((NSLICE, D), jnp.float32)
    b2 = b.reshape(1, D).astype(jnp.float32)

    traj = traj_seqs.astype(jnp.int32)
    is_pad = (traj < 0).astype(jnp.int32)
    mask = jnp.cumsum(is_pad, axis=1) == 0
    zrows = (jnp.arange(T, dtype=jnp.int32) % (NP - N)).reshape(B, S) + N
    idxsel = jnp.where(mask, jnp.clip(traj, 0, N - 1), zrows)
    idx2d = idxsel.reshape(TROWS, 128)

    degp = _deg_call(dst2d, w2d, zflat).reshape(NC, NP, 1)
    h_p, dis, g = _mmg_call(x_p, W, degp)
    acc = _msg_call(src2d, dst2d, w2d, g, zrow)
    emb = _emb_call(dis, h_p, acc, b2)
    out = _traj_call(emb, idx2d)

    return out.reshape(B, S, D), mask


# speedup vs baseline: 1.3305x; 1.0155x over previous
import jax
import jax.numpy as jnp
from jax import lax
from jax.experimental import pallas as pl
from jax.experimental.pallas import tpu as pltpu
from jax.experimental.pallas import tpu_sc as plsc

N = 10000
NP = 10240
E = 320000
EROWS = 2560
EP = EROWS * 128
B, S = 16, 2048
T = B * S
TROWS = T // 128
D = 128
NC, NS = 2, 16
NW = NC * NS
RPT = EROWS // NW
NSLICE = NP // NS

_mesh = lambda: plsc.VectorSubcoreMesh(
    core_axis_name="c", subcore_axis_name="s", num_cores=NC, num_subcores=NS)


def _deg_body(dst_hbm, w_hbm, zflat_hbm, deg_out, idxb, wb, sv, degsp):
    cid = lax.axis_index("c")
    sid = lax.axis_index("s")
    wid = cid * NS + sid
    pltpu.sync_copy(zflat_hbm, sv)
    pltpu.sync_copy(sv, degsp.at[pl.ds(sid * NSLICE, NSLICE)])
    plsc.subcore_barrier()

    def chunk(i, _):
        r0 = wid * RPT + i * 16
        pltpu.sync_copy(dst_hbm.at[pl.ds(r0, 16)], idxb)
        pltpu.sync_copy(w_hbm.at[pl.ds(r0, 16)], wb)
        for j in range(16):
            pltpu.sync_copy(wb.at[j], degsp.at[idxb.at[j]], add=True)
        return _

    lax.fori_loop(0, RPT // 16, chunk, None)
    plsc.subcore_barrier()
    pltpu.sync_copy(degsp.at[pl.ds(sid * NSLICE, NSLICE)], sv)
    pltpu.sync_copy(sv, deg_out.at[cid].at[pl.ds(sid * NSLICE, NSLICE)])


def _deg_call(dst2d, w2d, zflat):
    return pl.kernel(
        _deg_body,
        out_type=jax.ShapeDtypeStruct((NC, NP), jnp.float32),
        mesh=_mesh(),
        scratch_types=[
            pltpu.VMEM((16, 128), jnp.int32),
            pltpu.VMEM((16, 128), jnp.float32),
            pltpu.VMEM((NSLICE,), jnp.float32),
            pltpu.VMEM_SHARED((NP,), jnp.float32),
        ],
    )(dst2d, w2d, zflat)


NB = 2
GR = 128
GPT = EROWS // NW


def _msg_body(comb_hbm, w_hbm, g_hbm, zrow_hbm, acc_out,
              cb, wb, rows, gsem, ssem, accsp):
    cid = lax.axis_index("c")
    sid = lax.axis_index("s")
    wid = cid * NS + sid
    pltpu.sync_copy(zrow_hbm, accsp.at[pl.ds(sid * NSLICE, NSLICE)])
    plsc.subcore_barrier()

    def chunk(i, _):
        gc = wid * (GPT // 8) + i
        pltpu.sync_copy(comb_hbm.at[gc], cb)
        pltpu.sync_copy(w_hbm.at[pl.ds(gc * 8, 8)], wb)
        gd = [None] * 8
        sd = [None] * 8
        for j in range(NB):
            gd[j] = pltpu.async_copy(g_hbm.at[cb.at[j]], rows.at[j], gsem[j])
        for j in range(8):
            b = j % NB
            gd[j].wait()
            if NB <= j + 1 < 8:
                sd[j + 1 - NB].wait()
                gd[j + 1] = pltpu.async_copy(
                    g_hbm.at[cb.at[j + 1]], rows.at[(j + 1) % NB],
                    gsem[(j + 1) % NB])

            def scale_rows(gidx, _c):
                wv = wb[j, pl.ds(gidx * 16, 16)]
                for k in range(16):
                    bw = lax.broadcast(wv[k], (16,))
                    r = gidx * 16 + k
                    for q in range(8):
                        sl = pl.ds(q * 16, 16)
                        rows[b, r, sl] = rows[b, r, sl] * bw
                return _c

            lax.fori_loop(0, GR // 16, scale_rows, None)
            sd[j] = pltpu.async_copy(rows.at[b], accsp.at[cb.at[8 + j]],
                                     ssem[b], add=True)
        for j in range(8 - NB, 8):
            sd[j].wait()
        return _

    lax.fori_loop(0, GPT // 8, chunk, None)
    plsc.subcore_barrier()
    pltpu.sync_copy(accsp.at[pl.ds(sid * NSLICE, NSLICE)],
                    acc_out.at[cid].at[pl.ds(sid * NSLICE, NSLICE)])


def _msg_call(comb, w2d, g, zrow):
    return pl.kernel(
        _msg_body,
        out_type=jax.ShapeDtypeStruct((NC, NP, D), jnp.float32),
        mesh=_mesh(),
        scratch_types=[
            pltpu.VMEM((16, GR), jnp.int32),
            pltpu.VMEM((8, GR), jnp.float32),
            pltpu.VMEM((NB, GR, D), jnp.float32),
            [pltpu.SemaphoreType.DMA] * NB,
            [pltpu.SemaphoreType.DMA] * NB,
            pltpu.VMEM_SHARED((NP, D), jnp.float32),
        ],
    )(comb, w2d, g, zrow)


def _traj_body(emb_hbm, idx_hbm, out_hbm, idxb, rows, gsem, wsem):
    cid = lax.axis_index("c")
    sid = lax.axis_index("s")
    wid = cid * NS + sid
    rpt = TROWS // NW
    pltpu.sync_copy(idx_hbm.at[pl.ds(wid * rpt, rpt)], idxb)
    nb = 4
    gd = [None] * rpt
    wd = [None] * rpt
    for j in range(nb):
        gd[j] = pltpu.async_copy(emb_hbm.at[idxb.at[j]], rows.at[j], gsem[j])
    for j in range(rpt):
        b = j % nb
        gd[j].wait()
        if nb <= j + 1 < rpt:
            wd[j + 1 - nb].wait()
            gd[j + 1] = pltpu.async_copy(
                emb_hbm.at[idxb.at[j + 1]], rows.at[(j + 1) % nb],
                gsem[(j + 1) % nb])
        wd[j] = pltpu.async_copy(
            rows.at[b], out_hbm.at[pl.ds((wid * rpt + j) * 128, 128)], wsem[b])
    for j in range(rpt - nb, rpt):
        wd[j].wait()


def _traj_call(emb, idx2d):
    return pl.kernel(
        _traj_body,
        out_type=jax.ShapeDtypeStruct((T, D), jnp.float32),
        mesh=_mesh(),
        scratch_types=[
            pltpu.VMEM((TROWS // NW, 128), jnp.int32),
            pltpu.VMEM((4, 128, D), jnp.float32),
            [pltpu.SemaphoreType.DMA] * 4,
            [pltpu.SemaphoreType.DMA] * 4,
        ],
    )(emb, idx2d)


def _mmg_body(x_ref, w_ref, deg_ref, h_ref, dis_ref, g_ref):
    h = jnp.dot(x_ref[...], w_ref[...], preferred_element_type=jnp.float32)
    deg = 1.0 + deg_ref[0] + deg_ref[1]
    dis = lax.rsqrt(deg)
    h_ref[...] = h
    dis_ref[...] = dis
    g_ref[...] = h * dis


def _mmg_call(x_p, W, degp):
    blk = NP // 8
    return pl.pallas_call(
        _mmg_body,
        grid=(8,),
        in_specs=[
            pl.BlockSpec((blk, D), lambda i: (i, 0)),
            pl.BlockSpec((D, D), lambda i: (0, 0)),
            pl.BlockSpec((NC, blk, 1), lambda i: (0, i, 0)),
        ],
        out_specs=[
            pl.BlockSpec((blk, D), lambda i: (i, 0)),
            pl.BlockSpec((blk, 1), lambda i: (i, 0)),
            pl.BlockSpec((blk, D), lambda i: (i, 0)),
        ],
        out_shape=[
            jax.ShapeDtypeStruct((NP, D), jnp.float32),
            jax.ShapeDtypeStruct((NP, 1), jnp.float32),
            jax.ShapeDtypeStruct((NP, D), jnp.float32),
        ],
    )(x_p, W, degp)


def _emb_body(dis_ref, h_ref, acc_ref, b_ref, o_ref):
    i = pl.program_id(0)
    blk = NP // 8
    dis = dis_ref[...]
    s = acc_ref[0] + acc_ref[1]
    v = dis * s + (dis * dis) * h_ref[...] + b_ref[...]
    v = jnp.maximum(v, 0.0)
    row = i * blk + lax.broadcasted_iota(jnp.int32, (blk, D), 0)
    o_ref[...] = jnp.where(row < N, v, 0.0)


def _emb_call(dis, h_p, acc, b2):
    blk = NP // 8
    return pl.pallas_call(
        _emb_body,
        grid=(8,),
        in_specs=[
            pl.BlockSpec((blk, 1), lambda i: (i, 0)),
            pl.BlockSpec((blk, D), lambda i: (i, 0)),
            pl.BlockSpec((NC, blk, D), lambda i: (0, i, 0)),
            pl.BlockSpec((1, D), lambda i: (0, 0)),
        ],
        out_specs=pl.BlockSpec((blk, D), lambda i: (i, 0)),
        out_shape=jax.ShapeDtypeStruct((NP, D), jnp.float32),
    )(dis, h_p, acc, b2)


def kernel(x, edge_index, edge_attr, traj_seqs, W, b):
    src = edge_index[0].astype(jnp.int32)
    dst = edge_index[1].astype(jnp.int32)
    w = edge_attr.astype(jnp.float32)
    pad = EP - E
    spread = (jnp.arange(pad, dtype=jnp.int32) % (NP - N)) + N
    src2d = jnp.concatenate([src, spread]).reshape(EROWS, 128)
    dst2d = jnp.concatenate([dst, spread]).reshape(EROWS, 128)
    w2d = jnp.pad(w, (0, pad)).reshape(EROWS, 128)
    nchunk = EROWS // 8
    comb = jnp.concatenate(
        [src2d.reshape(nchunk, 8, 128),
         dst2d.reshape(nchunk, 8, 128)], axis=1)

    x_p = jnp.pad(x, ((0, NP - N), (0, 0)))
    zflat = jnp.zeros((NSLICE,), jnp.float32)
    zrow = jnp.zeros((NSLICE, D), jnp.float32)
    b2 = b.reshape(1, D).astype(jnp.float32)

    traj = traj_seqs.astype(jnp.int32)
    is_pad = (traj < 0).astype(jnp.int32)
    mask = jnp.cumsum(is_pad, axis=1) == 0
    zrows = (jnp.arange(T, dtype=jnp.int32) % (NP - N)).reshape(B, S) + N
    idxsel = jnp.where(mask, jnp.clip(traj, 0, N - 1), zrows)
    idx2d = idxsel.reshape(TROWS, 128)

    degp = _deg_call(dst2d, w2d, zflat).reshape(NC, NP, 1)
    h_p, dis, g = _mmg_call(x_p, W, degp)
    acc = _msg_call(comb, w2d, g, zrow)
    emb = _emb_call(dis, h_p, acc, b2)
    out = _traj_call(emb, idx2d)

    return out.reshape(B, S, D), mask
